# baseline (speedup 1.0000x reference)
import functools

import jax
import jax.numpy as jnp
from jax import lax
from jax.experimental import pallas as pl
from jax.experimental.pallas import tpu as pltpu
from jax.experimental.pallas import tpu_sc as plsc

VOCAB = 1000000
D = 64
DP = 128
N = 4096 * 50
NC, NS = 2, 16
NW = NC * NS
N_PER_W = N // NW
CH = 400
STEPS = N_PER_W // CH
NGP = STEPS // 2

_mesh = plsc.VectorSubcoreMesh(core_axis_name="c", subcore_axis_name="s")

NTC = VOCAB // DP
TAIL = NTC * DP


@functools.partial(
    pl.kernel,
    mesh=_mesh,
    out_type=jax.ShapeDtypeStruct((VOCAB, DP), jnp.float32),
    scratch_types=[
        pltpu.VMEM((2, D, DP), jnp.float32),
        pltpu.VMEM((2, DP, DP), jnp.float32),
        pltpu.SemaphoreType.DMA,
        pltpu.SemaphoreType.DMA,
        pltpu.SemaphoreType.DMA,
        pltpu.SemaphoreType.DMA,
    ],
    compiler_params=pltpu.CompilerParams(needs_layout_passes=False),
)
def _fmt_table(lut_t_hbm, tail_hbm, tp_hbm, blk_v, rows_v,
               i0sem, i1sem, o0sem, o1sem):
    wid = lax.axis_index("s") * NC + lax.axis_index("c")
    iota = lax.iota(jnp.int32, 16)
    isems = (i0sem, i1sem)
    osems = (o0sem, o1sem)

    def fire_in(c, h):
        pltpu.async_copy(
            lut_t_hbm.at[:, pl.ds(c * DP, DP)], blk_v.at[h], isems[h])

    def wait_in(h):
        pltpu.make_async_copy(
            lut_t_hbm.at[:, pl.ds(0, DP)], blk_v.at[h], isems[h]).wait()

    def fire_out(c, h):
        pltpu.async_copy(
            rows_v.at[h], tp_hbm.at[pl.ds(c * DP, DP)], osems[h])

    def wait_out(h):
        pltpu.make_async_copy(
            rows_v.at[h], tp_hbm.at[pl.ds(0, DP)], osems[h]).wait()

    rowv = [iota + dgrp * 16 for dgrp in range(4)]

    def transpose(h):
        @plsc.parallel_loop(0, DP, unroll=16)
        def col(ci):
            cs = jnp.full((16,), ci, jnp.int32)
            for dgrp in range(4):
                v = plsc.load_gather(blk_v.at[h], [rowv[dgrp], cs])
                rows_v[h, ci, pl.ds(dgrp * 16, 16)] = v

    nk = (NTC + NW - 1) // NW

    def guarded(k, fn):
        @pl.when(wid + NW * k < NTC)
        def _():
            fn(wid + NW * k)

    guarded(0, lambda c: fire_in(c, 0))
    guarded(1, lambda c: fire_in(c, 1))

    def step(k, _):
        h = 0
        for off in range(2):
            kk = 2 * k + off
            hh = off

            @pl.when(wid + NW * kk < NTC)
            def _(kk=kk, hh=hh):
                c = wid + NW * kk
                wait_in(hh)

                @pl.when(kk >= 2)
                def _():
                    wait_out(hh)

                transpose(hh)
                fire_out(c, hh)

                @pl.when(wid + NW * (kk + 2) < NTC)
                def _():
                    fire_in(wid + NW * (kk + 2), hh)
        return _

    lax.fori_loop(0, (nk + 1) // 2, step, None)

    @pl.when(wid + NW * (nk - 1) < NTC)
    def _():
        wait_out((nk - 1) % 2)

    @pl.when(wid + NW * (nk - 2) < NTC)
    def _():
        wait_out((nk - 2) % 2)

    @pl.when(wid == 0)
    def _():
        pltpu.sync_copy(tail_hbm, blk_v.at[0, :, :])
        pltpu.sync_copy(blk_v.at[0, :, :], tp_hbm.at[pl.ds(TAIL, D)])


@functools.partial(
    pl.kernel,
    mesh=_mesh,
    out_type=jax.ShapeDtypeStruct((4096, 50, D), jnp.float32),
    scratch_types=[
        pltpu.VMEM((STEPS, CH), jnp.int32),
        pltpu.VMEM((2, CH, DP), jnp.float32),
        pltpu.SemaphoreType.DMA,
        pltpu.SemaphoreType.DMA,
        pltpu.SemaphoreType.DMA,
        pltpu.SemaphoreType.DMA,
    ],
    compiler_params=pltpu.CompilerParams(use_tc_tiling_on_sc=False),
)
def _emb_lookup(idx_hbm, table_hbm, out_hbm, idx_v, rows_v,
                g0sem, g1sem, s0sem, s1sem):
    wid = lax.axis_index("s") * NC + lax.axis_index("c")
    bbase = wid * (N_PER_W // 50)
    bpg = CH // 50
    pltpu.sync_copy(idx_hbm.at[wid], idx_v)

    def fire(g, h, sem):
        pltpu.async_copy(table_hbm.at[idx_v.at[g]], rows_v.at[h], sem)

    def drain_gather(h, sem):
        pltpu.make_async_copy(
            table_hbm.at[idx_v.at[0]], rows_v.at[h], sem).wait()

    def scatter(g, h, sem):
        for j in range(bpg):
            pltpu.async_copy(
                rows_v.at[h, pl.ds(j * 50, 50), pl.ds(0, D)],
                out_hbm.at[bbase + g * bpg + j], sem)

    def drain_scatter(h, sem):
        for j in range(bpg):
            pltpu.make_async_copy(
                rows_v.at[h, pl.ds(j * 50, 50), pl.ds(0, D)],
                out_hbm.at[bbase], sem).wait()

    fire(0, 0, g0sem)

    def pair(p, _):
        g0 = 2 * p
        g1 = g0 + 1

        @pl.when(p > 0)
        def _():
            drain_scatter(1, s1sem)

        fire(g1, 1, g1sem)
        drain_gather(0, g0sem)
        scatter(g0, 0, s0sem)

        @pl.when(p + 1 < NGP)
        def _():
            drain_scatter(0, s0sem)
            fire(g0 + 2, 0, g0sem)

        drain_gather(1, g1sem)
        scatter(g1, 1, s1sem)
        return _

    lax.fori_loop(0, NGP, pair, None)
    drain_scatter(0, s0sem)
    drain_scatter(1, s1sem)


def kernel(x, lut):
    idx = x.reshape(N).astype(jnp.int32).reshape(NW, STEPS, CH)
    lut_t = lut.T
    tail_p = jnp.pad(lut[TAIL:, :], ((0, 0), (0, DP - D)))
    lut_p = _fmt_table(lut_t, tail_p)
    return _emb_lookup(idx, lut_p)


# speedup vs baseline: 1.3565x; 1.3565x over previous
import functools

import jax
import jax.numpy as jnp
from jax import lax
from jax.experimental import pallas as pl
from jax.experimental.pallas import tpu as pltpu
from jax.experimental.pallas import tpu_sc as plsc

VOCAB = 1000000
D = 64
DP = 128
N = 4096 * 50
NC, NS = 2, 16
NW = NC * NS
N_PER_W = N // NW
CH = 400
STEPS = N_PER_W // CH
NGP = STEPS // 2

_mesh = plsc.VectorSubcoreMesh(core_axis_name="c", subcore_axis_name="s")

@functools.partial(
    pl.kernel,
    mesh=_mesh,
    out_type=jax.ShapeDtypeStruct((4096, 50, D), jnp.float32),
    scratch_types=[
        pltpu.VMEM((STEPS, CH), jnp.int32),
        pltpu.VMEM((2, CH, DP), jnp.float32),
        pltpu.SemaphoreType.DMA,
        pltpu.SemaphoreType.DMA,
        pltpu.SemaphoreType.DMA,
        pltpu.SemaphoreType.DMA,
    ],
    compiler_params=pltpu.CompilerParams(use_tc_tiling_on_sc=False),
)
def _emb_lookup(idx_hbm, table_hbm, out_hbm, idx_v, rows_v,
                g0sem, g1sem, s0sem, s1sem):
    wid = lax.axis_index("s") * NC + lax.axis_index("c")
    bbase = wid * (N_PER_W // 50)
    bpg = CH // 50
    pltpu.sync_copy(idx_hbm.at[wid], idx_v)

    def fire(g, h, sem):
        pltpu.async_copy(table_hbm.at[idx_v.at[g]], rows_v.at[h], sem)

    def drain_gather(h, sem):
        pltpu.make_async_copy(
            table_hbm.at[idx_v.at[0]], rows_v.at[h], sem).wait()

    def scatter(g, h, sem):
        for j in range(bpg):
            pltpu.async_copy(
                rows_v.at[h, pl.ds(j * 50, 50), pl.ds(0, D)],
                out_hbm.at[bbase + g * bpg + j], sem)

    def drain_scatter(h, sem):
        for j in range(bpg):
            pltpu.make_async_copy(
                rows_v.at[h, pl.ds(j * 50, 50), pl.ds(0, D)],
                out_hbm.at[bbase], sem).wait()

    fire(0, 0, g0sem)

    def pair(p, _):
        g0 = 2 * p
        g1 = g0 + 1

        @pl.when(p > 0)
        def _():
            drain_scatter(1, s1sem)

        fire(g1, 1, g1sem)
        drain_gather(0, g0sem)
        scatter(g0, 0, s0sem)

        @pl.when(p + 1 < NGP)
        def _():
            drain_scatter(0, s0sem)
            fire(g0 + 2, 0, g0sem)

        drain_gather(1, g1sem)
        scatter(g1, 1, s1sem)
        return _

    lax.fori_loop(0, NGP, pair, None)
    drain_scatter(0, s0sem)
    drain_scatter(1, s1sem)


def kernel(x, lut):
    idx = x.reshape(N).astype(jnp.int32).reshape(NW, STEPS, CH)
    lut_p = jnp.pad(lut, ((0, 0), (0, DP - D)))
    return _emb_lookup(idx, lut_p)
